# baseline (speedup 1.0000x reference)
import functools

import jax
import jax.numpy as jnp
from jax import lax
from jax.experimental import pallas as pl
from jax.experimental.pallas import tpu as pltpu
from jax.experimental.pallas import tpu_sc as plsc

N = 10000
E = 320000
D = 128
NG = 64
NC = 2
NS = 16
NW = NC * NS
EPW = E // NW
K = 40
IC = 25
NCHK = EPW // (IC * K)
NP = 10240
RPT = NP // NS
ZR = 16
NB = 1000
GN = N // NB


def _mm(a, b):
    return jnp.dot(a.astype(jnp.bfloat16), b.astype(jnp.bfloat16),
                   preferred_element_type=jnp.float32)


def _encode_ab_body(x_ref, w_ref, b_ref, ws_ref, wd_ref, bm_ref,
                    h_ref, a_ref, b2_ref):
    h = jnp.maximum(_mm(x_ref[...], w_ref[...]) + b_ref[...], 0.0)
    h_ref[...] = h
    a_ref[...] = _mm(h, ws_ref[...]) + bm_ref[...]
    b2_ref[...] = _mm(h, wd_ref[...])


def _encode_ab(x, W, b, Ws, Wd, bm):
    return pl.pallas_call(
        _encode_ab_body,
        grid=(GN,),
        in_specs=[
            pl.BlockSpec((NB, D), lambda i: (i, 0)),
            pl.BlockSpec((D, D), lambda i: (0, 0)),
            pl.BlockSpec((1, D), lambda i: (0, 0)),
            pl.BlockSpec((D, D), lambda i: (0, 0)),
            pl.BlockSpec((D, D), lambda i: (0, 0)),
            pl.BlockSpec((1, D), lambda i: (0, 0)),
        ],
        out_specs=[pl.BlockSpec((NB, D), lambda i: (i, 0))] * 3,
        out_shape=[jax.ShapeDtypeStruct((N, D), jnp.float32)] * 3,
    )(x, W, b.reshape(1, D), Ws, Wd, bm.reshape(1, D))


EB = 2000


def _edgec_body(ea_ref, we_ref, be_ref, wm_ref, o_ref):
    e = jnp.maximum(_mm(ea_ref[...], we_ref[...]) + be_ref[...], 0.0)
    o_ref[...] = _mm(e, wm_ref[...])


def _edge_c(edge_attr, W_edge, b_edge, Wm):
    return pl.pallas_call(
        _edgec_body,
        grid=(E // EB,),
        in_specs=[
            pl.BlockSpec((EB, 16), lambda i: (i, 0)),
            pl.BlockSpec((16, D), lambda i: (0, 0)),
            pl.BlockSpec((1, D), lambda i: (0, 0)),
            pl.BlockSpec((D, D), lambda i: (0, 0)),
        ],
        out_specs=pl.BlockSpec((EB, D), lambda i: (i, 0)),
        out_shape=jax.ShapeDtypeStruct((E, D), jnp.float32),
    )(edge_attr, W_edge, b_edge.reshape(1, D), Wm)


def _upd_body(h_ref, g0_ref, g1_ref, u1_ref, u2_ref, bu_ref, o_ref):
    agg = g0_ref[...] + g1_ref[...]
    o_ref[...] = jnp.maximum(
        _mm(h_ref[...], u1_ref[...]) + _mm(agg, u2_ref[...]) + bu_ref[...], 0.0)


def _upd(h, g0, g1, U1, U2, bu):
    return pl.pallas_call(
        _upd_body,
        grid=(GN,),
        in_specs=[
            pl.BlockSpec((NB, D), lambda i: (i, 0)),
            pl.BlockSpec((NB, D), lambda i: (i, 0)),
            pl.BlockSpec((NB, D), lambda i: (i, 0)),
            pl.BlockSpec((D, D), lambda i: (0, 0)),
            pl.BlockSpec((D, D), lambda i: (0, 0)),
            pl.BlockSpec((1, D), lambda i: (0, 0)),
        ],
        out_specs=pl.BlockSpec((NB, D), lambda i: (i, 0)),
        out_shape=jax.ShapeDtypeStruct((N, D), jnp.float32),
    )(h, g0, g1, U1, U2, bu.reshape(1, D))


def _upd_ab_body(h_ref, g0_ref, g1_ref, u1_ref, u2_ref, bu_ref,
                 ws_ref, wd_ref, bm_ref, o_ref, a_ref, b2_ref):
    agg = g0_ref[...] + g1_ref[...]
    h = jnp.maximum(
        _mm(h_ref[...], u1_ref[...]) + _mm(agg, u2_ref[...]) + bu_ref[...], 0.0)
    o_ref[...] = h
    a_ref[...] = _mm(h, ws_ref[...]) + bm_ref[...]
    b2_ref[...] = _mm(h, wd_ref[...])


def _upd_ab(h, g0, g1, U1, U2, bu, Ws, Wd, bm):
    return pl.pallas_call(
        _upd_ab_body,
        grid=(GN,),
        in_specs=[
            pl.BlockSpec((NB, D), lambda i: (i, 0)),
            pl.BlockSpec((NB, D), lambda i: (i, 0)),
            pl.BlockSpec((NB, D), lambda i: (i, 0)),
            pl.BlockSpec((D, D), lambda i: (0, 0)),
            pl.BlockSpec((D, D), lambda i: (0, 0)),
            pl.BlockSpec((1, D), lambda i: (0, 0)),
            pl.BlockSpec((D, D), lambda i: (0, 0)),
            pl.BlockSpec((D, D), lambda i: (0, 0)),
            pl.BlockSpec((1, D), lambda i: (0, 0)),
        ],
        out_specs=[pl.BlockSpec((NB, D), lambda i: (i, 0))] * 3,
        out_shape=[jax.ShapeDtypeStruct((N, D), jnp.float32)] * 3,
    )(h, g0, g1, U1, U2, bu.reshape(1, D), Ws, Wd, bm.reshape(1, D))


def _sc_edge(A, B, C4, src3, dst3):
    mesh = plsc.VectorSubcoreMesh(core_axis_name="c", subcore_axis_name="s")

    @functools.partial(
        pl.kernel,
        out_type=jax.ShapeDtypeStruct((NC, NP, D), jnp.float32),
        mesh=mesh,
        scratch_types=[
            pltpu.VMEM_SHARED((NP, D), jnp.float32),
            pltpu.VMEM((IC, K), jnp.int32),
            pltpu.VMEM((IC, K), jnp.int32),
            pltpu.VMEM((3, K, D), jnp.float32),
            pltpu.VMEM((2, K, D), jnp.float32),
            pltpu.VMEM((2, K, D), jnp.float32),
            pltpu.VMEM((ZR, D), jnp.float32),
            pltpu.SemaphoreType.DMA,
            pltpu.SemaphoreType.DMA,
            pltpu.SemaphoreType.DMA,
        ],
    )
    def k(a_hbm, b_hbm, c_hbm, src_hbm, dst_hbm, out_hbm,
          agg_sh, src_v, dst_v, bufA, bufB, bufC, zbuf,
          sem0, sem1, zsem):
        c = lax.axis_index("c")
        s = lax.axis_index("s")
        w = c * NS + s
        sems = (sem0, sem1)

        zero16 = jnp.zeros((16,), jnp.float32)

        @plsc.parallel_loop(0, ZR, unroll=1)
        def _(r):
            for cc in range(D // 16):
                zbuf[r, pl.ds(cc * 16, 16)] = zero16

        for t in range(RPT // ZR):
            pltpu.async_copy(zbuf, agg_sh.at[pl.ds(s * RPT + t * ZR, ZR)],
                             zsem)
        for t in range(RPT // ZR):
            pltpu.make_async_copy(
                zbuf, agg_sh.at[pl.ds(s * RPT, ZR)], zsem).wait()
        plsc.subcore_barrier()

        def issue(ch, j, jm3, jm2):
            pltpu.async_copy(a_hbm.at[src_v.at[j]], bufA.at[jm3], sems[jm2])
            pltpu.async_copy(b_hbm.at[dst_v.at[j]], bufB.at[jm2], sems[jm2])
            pltpu.async_copy(c_hbm.at[w, ch, j], bufC.at[jm2], sems[jm2])

        def drain(jm2):
            for buf in (bufA.at[0], bufB.at[0], bufC.at[0]):
                pltpu.make_async_copy(
                    a_hbm.at[pl.ds(0, K)], buf, sems[jm2]).wait()

        def window(ch, j, jm3, jm2, last):
            if not last:
                nm3, nm2 = (jm3 + 1) % 3, 1 - jm2
                issue(ch, j + 1, nm3, nm2)
            drain(jm2)
            BA, BB, BC = bufA.at[jm3], bufB.at[jm2], bufC.at[jm2]

            @plsc.parallel_loop(0, K, unroll=2)
            def _(r):
                for cc in range(D // 16):
                    sl = pl.ds(cc * 16, 16)
                    BA[r, sl] = jnp.maximum(BA[r, sl] + BB[r, sl] + BC[r, sl],
                                            0.0)

            pltpu.sync_copy(bufA.at[jm3], agg_sh.at[dst_v.at[j]], add=True)

        @pl.loop(0, NCHK)
        def _(ch0):
            pltpu.sync_copy(src_hbm.at[w, ch0], src_v)
            pltpu.sync_copy(dst_hbm.at[w, ch0], dst_v)
            issue(ch0, 0, 0, 0)
            for j in range(6):
                window(ch0, j, j % 3, j % 2, False)

            @pl.loop(6, IC - 1, step=6)
            def _(j6):
                for r in range(6):
                    window(ch0, j6 + r, r % 3, r % 2, False)

            window(ch0, IC - 1, 0, 0, True)

        plsc.subcore_barrier()
        pltpu.sync_copy(agg_sh.at[pl.ds(s * RPT, RPT)],
                        out_hbm.at[c, pl.ds(s * RPT, RPT)])

    return k(A, B, C4, src3, dst3)


def _pool_body(bt_ref, h_ref, wgp_ref, bgp_ref, w1_ref, b1_ref, w2_ref,
               b2_ref, w3_ref, b3_ref, o_ref, gsum, cnt):
    i = pl.program_id(0)

    @pl.when(i == 0)
    def _():
        gsum[...] = jnp.zeros_like(gsum)
        cnt[...] = jnp.zeros_like(cnt)

    bt = bt_ref[0]
    oh = (lax.broadcasted_iota(jnp.int32, (NG, NB), 0) == bt).astype(
        jnp.float32)
    gsum[...] += jnp.dot(oh, h_ref[...], preferred_element_type=jnp.float32,
                         precision=lax.Precision.HIGHEST)
    cnt[...] += jnp.broadcast_to(jnp.sum(oh, axis=1, keepdims=True), (NG, D))

    @pl.when(i == GN - 1)
    def _():
        gmean = gsum[...] / jnp.maximum(cnt[...], 1.0)
        g = jnp.maximum(_mm(gmean, wgp_ref[...]) + bgp_ref[...], 0.0)
        p = jnp.maximum(_mm(g, w1_ref[...]) + b1_ref[...], 0.0)
        p = jnp.maximum(_mm(p, w2_ref[...]) + b2_ref[...], 0.0)
        o_ref[...] = _mm(p, w3_ref[...]) + b3_ref[...]


def _pool(batch3, h, W_gp, b_gp, W1g, b1, W2, b2, W3, b3):
    F2, F4 = W1g.shape[1], W2.shape[1]
    return pl.pallas_call(
        _pool_body,
        grid=(GN,),
        in_specs=[
            pl.BlockSpec((1, 1, NB), lambda i: (i, 0, 0)),
            pl.BlockSpec((NB, D), lambda i: (i, 0)),
            pl.BlockSpec((D, D), lambda i: (0, 0)),
            pl.BlockSpec((1, D), lambda i: (0, 0)),
            pl.BlockSpec((D, F2), lambda i: (0, 0)),
            pl.BlockSpec((1, F2), lambda i: (0, 0)),
            pl.BlockSpec((F2, F4), lambda i: (0, 0)),
            pl.BlockSpec((1, F4), lambda i: (0, 0)),
            pl.BlockSpec((F4, 1), lambda i: (0, 0)),
            pl.BlockSpec((1, 1), lambda i: (0, 0)),
        ],
        out_specs=pl.BlockSpec((NG, 1), lambda i: (0, 0)),
        out_shape=jax.ShapeDtypeStruct((NG, 1), jnp.float32),
        scratch_shapes=[
            pltpu.VMEM((NG, D), jnp.float32),
            pltpu.VMEM((NG, D), jnp.float32),
        ],
    )(batch3, h, W_gp, b_gp.reshape(1, D), W1g, b1.reshape(1, F2), W2,
      b2.reshape(1, F4), W3, b3.reshape(1, 1))


def kernel(x, edge_index, edge_attr, batch, W_node, b_node, W_edge, b_edge,
           W_msg, b_msg, W_upd, b_upd, W_gp, b_gp, W_p1, b_p1, W_p2, b_p2,
           W_p3, b_p3):
    src3 = edge_index[0].reshape(NW, NCHK, IC, K)
    dst3 = edge_index[1].reshape(NW, NCHK, IC, K)

    n_layers = W_msg.shape[0]
    Cs = [
        _edge_c(edge_attr, W_edge, b_edge,
                W_msg[i, 2 * D:3 * D, :]).reshape(NW, NCHK, IC, K, D)
        for i in range(n_layers)
    ]
    h, A, B = _encode_ab(x, W_node, b_node, W_msg[0, :D, :],
                         W_msg[0, D:2 * D, :], b_msg[0])
    for i in range(n_layers):
        agg2 = _sc_edge(A, B, Cs[i], src3, dst3)
        if i + 1 < n_layers:
            h, A, B = _upd_ab(h, agg2[0, :N], agg2[1, :N], W_upd[i, :D, :],
                              W_upd[i, D:, :], b_upd[i], W_msg[i + 1, :D, :],
                              W_msg[i + 1, D:2 * D, :], b_msg[i + 1])
        else:
            h = _upd(h, agg2[0, :N], agg2[1, :N], W_upd[i, :D, :],
                     W_upd[i, D:, :], b_upd[i])

    return _pool(batch.reshape(GN, 1, NB), h, W_gp, b_gp, W_p1[320:, :],
                 b_p1, W_p2, b_p2, W_p3, b_p3)


# speedup vs baseline: 1.0422x; 1.0422x over previous
import functools

import jax
import jax.numpy as jnp
from jax import lax
from jax.experimental import pallas as pl
from jax.experimental.pallas import tpu as pltpu
from jax.experimental.pallas import tpu_sc as plsc

N = 10000
E = 320000
D = 128
NG = 64
NC = 2
NS = 16
NW = NC * NS
EPW = E // NW
K = 40
IC = 25
NCHK = EPW // (IC * K)
NP = 10240
RPT = NP // NS
ZR = 16
NB = 1000
GN = N // NB


def _mm(a, b):
    return jnp.dot(a.astype(jnp.bfloat16), b.astype(jnp.bfloat16),
                   preferred_element_type=jnp.float32)


def _encode_ab_body(x_ref, w_ref, b_ref, ws_ref, wd_ref, bm_ref,
                    h_ref, a_ref, b2_ref):
    h = jnp.maximum(_mm(x_ref[...], w_ref[...]) + b_ref[...], 0.0)
    h_ref[...] = h
    a_ref[...] = _mm(h, ws_ref[...]) + bm_ref[...]
    b2_ref[...] = _mm(h, wd_ref[...])


def _encode_ab(x, W, b, Ws, Wd, bm):
    return pl.pallas_call(
        _encode_ab_body,
        grid=(GN,),
        in_specs=[
            pl.BlockSpec((NB, D), lambda i: (i, 0)),
            pl.BlockSpec((D, D), lambda i: (0, 0)),
            pl.BlockSpec((1, D), lambda i: (0, 0)),
            pl.BlockSpec((D, D), lambda i: (0, 0)),
            pl.BlockSpec((D, D), lambda i: (0, 0)),
            pl.BlockSpec((1, D), lambda i: (0, 0)),
        ],
        out_specs=[pl.BlockSpec((NB, D), lambda i: (i, 0))] * 3,
        out_shape=[jax.ShapeDtypeStruct((N, D), jnp.float32)] * 3,
    )(x, W, b.reshape(1, D), Ws, Wd, bm.reshape(1, D))


EB = 2000


def _edgec_body(ea_ref, we_ref, be_ref, wm_ref, o_ref):
    e = jnp.maximum(_mm(ea_ref[...], we_ref[...]) + be_ref[...], 0.0)
    o_ref[...] = _mm(e, wm_ref[...])


def _edge_c(edge_attr, W_edge, b_edge, Wm):
    return pl.pallas_call(
        _edgec_body,
        grid=(E // EB,),
        in_specs=[
            pl.BlockSpec((EB, 16), lambda i: (i, 0)),
            pl.BlockSpec((16, D), lambda i: (0, 0)),
            pl.BlockSpec((1, D), lambda i: (0, 0)),
            pl.BlockSpec((D, D), lambda i: (0, 0)),
        ],
        out_specs=pl.BlockSpec((EB, D), lambda i: (i, 0)),
        out_shape=jax.ShapeDtypeStruct((E, D), jnp.float32),
    )(edge_attr, W_edge, b_edge.reshape(1, D), Wm)


def _upd_body(h_ref, g0_ref, g1_ref, u1_ref, u2_ref, bu_ref, o_ref):
    agg = g0_ref[...] + g1_ref[...]
    o_ref[...] = jnp.maximum(
        _mm(h_ref[...], u1_ref[...]) + _mm(agg, u2_ref[...]) + bu_ref[...], 0.0)


def _upd(h, g0, g1, U1, U2, bu):
    return pl.pallas_call(
        _upd_body,
        grid=(GN,),
        in_specs=[
            pl.BlockSpec((NB, D), lambda i: (i, 0)),
            pl.BlockSpec((NB, D), lambda i: (i, 0)),
            pl.BlockSpec((NB, D), lambda i: (i, 0)),
            pl.BlockSpec((D, D), lambda i: (0, 0)),
            pl.BlockSpec((D, D), lambda i: (0, 0)),
            pl.BlockSpec((1, D), lambda i: (0, 0)),
        ],
        out_specs=pl.BlockSpec((NB, D), lambda i: (i, 0)),
        out_shape=jax.ShapeDtypeStruct((N, D), jnp.float32),
    )(h, g0, g1, U1, U2, bu.reshape(1, D))


def _upd_ab_body(h_ref, g0_ref, g1_ref, u1_ref, u2_ref, bu_ref,
                 ws_ref, wd_ref, bm_ref, o_ref, a_ref, b2_ref):
    agg = g0_ref[...] + g1_ref[...]
    h = jnp.maximum(
        _mm(h_ref[...], u1_ref[...]) + _mm(agg, u2_ref[...]) + bu_ref[...], 0.0)
    o_ref[...] = h
    a_ref[...] = _mm(h, ws_ref[...]) + bm_ref[...]
    b2_ref[...] = _mm(h, wd_ref[...])


def _upd_ab(h, g0, g1, U1, U2, bu, Ws, Wd, bm):
    return pl.pallas_call(
        _upd_ab_body,
        grid=(GN,),
        in_specs=[
            pl.BlockSpec((NB, D), lambda i: (i, 0)),
            pl.BlockSpec((NB, D), lambda i: (i, 0)),
            pl.BlockSpec((NB, D), lambda i: (i, 0)),
            pl.BlockSpec((D, D), lambda i: (0, 0)),
            pl.BlockSpec((D, D), lambda i: (0, 0)),
            pl.BlockSpec((1, D), lambda i: (0, 0)),
            pl.BlockSpec((D, D), lambda i: (0, 0)),
            pl.BlockSpec((D, D), lambda i: (0, 0)),
            pl.BlockSpec((1, D), lambda i: (0, 0)),
        ],
        out_specs=[pl.BlockSpec((NB, D), lambda i: (i, 0))] * 3,
        out_shape=[jax.ShapeDtypeStruct((N, D), jnp.float32)] * 3,
    )(h, g0, g1, U1, U2, bu.reshape(1, D), Ws, Wd, bm.reshape(1, D))


def _sc_edge(A, B, C4, src3, dst3):
    mesh = plsc.VectorSubcoreMesh(core_axis_name="c", subcore_axis_name="s")

    @functools.partial(
        pl.kernel,
        out_type=jax.ShapeDtypeStruct((NC, NP, D), jnp.float32),
        mesh=mesh,
        scratch_types=[
            pltpu.VMEM_SHARED((NP, D), jnp.float32),
            pltpu.VMEM((IC, K), jnp.int32),
            pltpu.VMEM((IC, K), jnp.int32),
            pltpu.VMEM((3, K, D), jnp.float32),
            pltpu.VMEM((2, K, D), jnp.float32),
            pltpu.VMEM((2, K, D), jnp.float32),
            pltpu.VMEM((ZR, D), jnp.float32),
            pltpu.SemaphoreType.DMA,
            pltpu.SemaphoreType.DMA,
            pltpu.SemaphoreType.DMA,
            pltpu.SemaphoreType.DMA,
            pltpu.SemaphoreType.DMA,
            pltpu.SemaphoreType.DMA,
        ],
    )
    def k(a_hbm, b_hbm, c_hbm, src_hbm, dst_hbm, out_hbm,
          agg_sh, src_v, dst_v, bufA, bufB, bufC, zbuf,
          sem0, sem1, zsem, ssem0, ssem1, ssem2):
        c = lax.axis_index("c")
        s = lax.axis_index("s")
        w = c * NS + s
        sems = (sem0, sem1)
        ssems = (ssem0, ssem1, ssem2)

        zero16 = jnp.zeros((16,), jnp.float32)

        @plsc.parallel_loop(0, ZR, unroll=1)
        def _(r):
            for cc in range(D // 16):
                zbuf[r, pl.ds(cc * 16, 16)] = zero16

        for t in range(RPT // ZR):
            pltpu.async_copy(zbuf, agg_sh.at[pl.ds(s * RPT + t * ZR, ZR)],
                             zsem)
        for t in range(RPT // ZR):
            pltpu.make_async_copy(
                zbuf, agg_sh.at[pl.ds(s * RPT, ZR)], zsem).wait()
        plsc.subcore_barrier()

        def issue(ch, j, jm3, jm2):
            pltpu.async_copy(a_hbm.at[src_v.at[j]], bufA.at[jm3], sems[jm2])
            pltpu.async_copy(b_hbm.at[dst_v.at[j]], bufB.at[jm2], sems[jm2])
            pltpu.async_copy(c_hbm.at[w, ch, j], bufC.at[jm2], sems[jm2])

        def drain(jm2):
            for buf in (bufA.at[0], bufB.at[0], bufC.at[0]):
                pltpu.make_async_copy(
                    a_hbm.at[pl.ds(0, K)], buf, sems[jm2]).wait()

        def swait(slot):
            pltpu.make_async_copy(
                bufA.at[0], agg_sh.at[pl.ds(0, K)], ssems[slot]).wait()

        def window(ch, j, jm3, jm2, wait_reuse, last):
            if not last:
                nm3, nm2 = (jm3 + 1) % 3, 1 - jm2
                if wait_reuse:
                    swait(nm3)
                issue(ch, j + 1, nm3, nm2)
            drain(jm2)
            BA, BB, BC = bufA.at[jm3], bufB.at[jm2], bufC.at[jm2]

            @plsc.parallel_loop(0, K, unroll=2)
            def _(r):
                for cc in range(D // 16):
                    sl = pl.ds(cc * 16, 16)
                    BA[r, sl] = jnp.maximum(BA[r, sl] + BB[r, sl] + BC[r, sl],
                                            0.0)

            pltpu.async_copy(bufA.at[jm3], agg_sh.at[dst_v.at[j]], ssems[jm3],
                             add=True)

        @pl.loop(0, NCHK)
        def _(ch0):
            pltpu.sync_copy(src_hbm.at[w, ch0], src_v)
            pltpu.sync_copy(dst_hbm.at[w, ch0], dst_v)
            issue(ch0, 0, 0, 0)
            for j in range(6):
                window(ch0, j, j % 3, j % 2, j >= 2, False)

            @pl.loop(6, IC - 1, step=6)
            def _(j6):
                for r in range(6):
                    window(ch0, j6 + r, r % 3, r % 2, True, False)

            window(ch0, IC - 1, 0, 0, False, True)
            swait(1)
            swait(2)
            swait(0)

        plsc.subcore_barrier()
        pltpu.sync_copy(agg_sh.at[pl.ds(s * RPT, RPT)],
                        out_hbm.at[c, pl.ds(s * RPT, RPT)])

    return k(A, B, C4, src3, dst3)


def _pool_body(bt_ref, h_ref, wgp_ref, bgp_ref, w1_ref, b1_ref, w2_ref,
               b2_ref, w3_ref, b3_ref, o_ref, gsum, cnt):
    i = pl.program_id(0)

    @pl.when(i == 0)
    def _():
        gsum[...] = jnp.zeros_like(gsum)
        cnt[...] = jnp.zeros_like(cnt)

    bt = bt_ref[0]
    oh = (lax.broadcasted_iota(jnp.int32, (NG, NB), 0) == bt).astype(
        jnp.float32)
    gsum[...] += jnp.dot(oh, h_ref[...], preferred_element_type=jnp.float32,
                         precision=lax.Precision.HIGHEST)
    cnt[...] += jnp.broadcast_to(jnp.sum(oh, axis=1, keepdims=True), (NG, D))

    @pl.when(i == GN - 1)
    def _():
        gmean = gsum[...] / jnp.maximum(cnt[...], 1.0)
        g = jnp.maximum(_mm(gmean, wgp_ref[...]) + bgp_ref[...], 0.0)
        p = jnp.maximum(_mm(g, w1_ref[...]) + b1_ref[...], 0.0)
        p = jnp.maximum(_mm(p, w2_ref[...]) + b2_ref[...], 0.0)
        o_ref[...] = _mm(p, w3_ref[...]) + b3_ref[...]


def _pool(batch3, h, W_gp, b_gp, W1g, b1, W2, b2, W3, b3):
    F2, F4 = W1g.shape[1], W2.shape[1]
    return pl.pallas_call(
        _pool_body,
        grid=(GN,),
        in_specs=[
            pl.BlockSpec((1, 1, NB), lambda i: (i, 0, 0)),
            pl.BlockSpec((NB, D), lambda i: (i, 0)),
            pl.BlockSpec((D, D), lambda i: (0, 0)),
            pl.BlockSpec((1, D), lambda i: (0, 0)),
            pl.BlockSpec((D, F2), lambda i: (0, 0)),
            pl.BlockSpec((1, F2), lambda i: (0, 0)),
            pl.BlockSpec((F2, F4), lambda i: (0, 0)),
            pl.BlockSpec((1, F4), lambda i: (0, 0)),
            pl.BlockSpec((F4, 1), lambda i: (0, 0)),
            pl.BlockSpec((1, 1), lambda i: (0, 0)),
        ],
        out_specs=pl.BlockSpec((NG, 1), lambda i: (0, 0)),
        out_shape=jax.ShapeDtypeStruct((NG, 1), jnp.float32),
        scratch_shapes=[
            pltpu.VMEM((NG, D), jnp.float32),
            pltpu.VMEM((NG, D), jnp.float32),
        ],
    )(batch3, h, W_gp, b_gp.reshape(1, D), W1g, b1.reshape(1, F2), W2,
      b2.reshape(1, F4), W3, b3.reshape(1, 1))


def kernel(x, edge_index, edge_attr, batch, W_node, b_node, W_edge, b_edge,
           W_msg, b_msg, W_upd, b_upd, W_gp, b_gp, W_p1, b_p1, W_p2, b_p2,
           W_p3, b_p3):
    src3 = edge_index[0].reshape(NW, NCHK, IC, K)
    dst3 = edge_index[1].reshape(NW, NCHK, IC, K)

    n_layers = W_msg.shape[0]
    Cs = [
        _edge_c(edge_attr, W_edge, b_edge,
                W_msg[i, 2 * D:3 * D, :]).reshape(NW, NCHK, IC, K, D)
        for i in range(n_layers)
    ]
    h, A, B = _encode_ab(x, W_node, b_node, W_msg[0, :D, :],
                         W_msg[0, D:2 * D, :], b_msg[0])
    for i in range(n_layers):
        agg2 = _sc_edge(A, B, Cs[i], src3, dst3)
        if i + 1 < n_layers:
            h, A, B = _upd_ab(h, agg2[0, :N], agg2[1, :N], W_upd[i, :D, :],
                              W_upd[i, D:, :], b_upd[i], W_msg[i + 1, :D, :],
                              W_msg[i + 1, D:2 * D, :], b_msg[i + 1])
        else:
            h = _upd(h, agg2[0, :N], agg2[1, :N], W_upd[i, :D, :],
                     W_upd[i, D:, :], b_upd[i])

    return _pool(batch.reshape(GN, 1, NB), h, W_gp, b_gp, W_p1[320:, :],
                 b_p1, W_p2, b_p2, W_p3, b_p3)
